# Initial kernel scaffold; baseline (speedup 1.0000x reference)
import jax
import jax.numpy as jnp
from jax.experimental import pallas as pl


def kernel(x, fc_w, fc_b, w1_0, w1_1, w1_2, w1_3, g1, beta1, w2_0, w2_1, w2_2, w2_3, g2, beta2, w3_0, w3_1, w3_2, w3_3, g3, beta3, w4_0, w4_1, w4_2, w4_3, b4):
    raise NotImplementedError("write your pallas kernel here")


# speedup vs baseline: 9.6979x; 9.6979x over previous
import functools

import jax
import jax.numpy as jnp
from jax.experimental import pallas as pl
from jax.experimental.pallas import tpu as pltpu

EPS = 1e-5
_PARITIES = ((0, 0), (0, 1), (1, 0), (1, 1))
_ALL_TAPS = tuple((dy, dx) for dy in (0, 1, 2) for dx in (0, 1, 2))

_CP = pltpu.CompilerParams(
    dimension_semantics=("parallel",),
    vmem_limit_bytes=48 * 1024 * 1024,
)


def _offs(p):
    return (0, 1, 2) if p == 0 else (1, 2)


def _ptaps(a, b):
    return tuple((dy, dx) for dy in _offs(a) for dx in _offs(b))


def _tap_blocks(w, a, b, cin):
    taps = _ptaps(a, b)
    wr = w.reshape(len(taps), cin, w.shape[-1])
    return {t: wr[i] for i, t in enumerate(taps)}


def _group_weight(blocks_list, taps, cin, cout_each):
    dt = next(iter(blocks_list[0].values())).dtype
    mats = []
    for t in taps:
        cols = [blocks.get(t, jnp.zeros((cin, cout_each), dt))
                for blocks in blocks_list]
        mats.append(jnp.concatenate(cols, axis=1) if len(cols) > 1 else cols[0])
    return jnp.stack(mats, axis=0)


def _w4_combined(w4s):
    blocks = [_tap_blocks(w4s[j], a, b, 64) for j, (a, b) in enumerate(_PARITIES)]
    mats = []
    for t in _ALL_TAPS:
        m = jnp.zeros((64, 128), jnp.bfloat16)
        for j in range(4):
            if t in blocks[j]:
                m = m.at[:, j:12:4].set(blocks[j][t][:, :3])
        mats.append(m)
    return jnp.stack(mats, axis=0)


def _conv_kernel(bt, h, w, cin, groups, bn, count_prev, pairs_prev, finale,
                 *refs):
    G = len(groups)
    k = 0
    x_ref = refs[k]; k += 1
    if bn:
        stats_ref, g_ref, be_ref = refs[k], refs[k + 1], refs[k + 2]; k += 3
    if finale:
        bias_ref = refs[k]; k += 1
    w_refs = refs[k:k + G]; k += G
    o_refs = refs[k:k + G]; k += G
    if not finale:
        so_ref = refs[k]; k += 1

    x = x_ref[...]
    if bn:
        tot = jnp.sum(stats_ref[...], axis=0)
        gp = tot.shape[0] // 2
        ssum = jnp.sum(tot[:gp], axis=0, keepdims=True)
        ssq = jnp.sum(tot[gp:], axis=0, keepdims=True)
        if pairs_prev:
            c = ssum.shape[1] // 2
            ssum = ssum[:, :c] + ssum[:, c:]
            ssq = ssq[:, :c] + ssq[:, c:]
        mean = ssum / count_prev
        var = jnp.maximum(ssq / count_prev - mean * mean, 0.0)
        scale = g_ref[...] * jax.lax.rsqrt(var + EPS)
        shift = be_ref[...] - mean * scale
        xf = (x.astype(jnp.float32) * scale.reshape(1, 1, 1, -1)
              + shift.reshape(1, 1, 1, -1))
        x = jnp.maximum(xf, 0.0).astype(jnp.bfloat16)

    zrow = jnp.zeros((bt, 1, w, cin), jnp.bfloat16)
    yp = jnp.concatenate([zrow, x, zrow], axis=1)
    zcol = jnp.zeros((bt, h + 2, 1, cin), jnp.bfloat16)
    yp = jnp.concatenate([zcol, yp, zcol], axis=2)

    r = bt * h * w
    sums, sqs = [], []
    for gi, taps in enumerate(groups):
        z = None
        for t, (dy, dx) in enumerate(taps):
            sl = yp[:, dy:dy + h, dx:dx + w, :].reshape(r, cin)
            zz = jnp.dot(sl, w_refs[gi][t], preferred_element_type=jnp.float32)
            z = zz if z is None else z + zz
        if finale:
            z = jnp.tanh(z + bias_ref[...])
            o_refs[gi][...] = z[:, :12]
        else:
            zc = z.astype(jnp.bfloat16)
            o_refs[gi][...] = zc
            zf = zc.astype(jnp.float32)
            sums.append(jnp.sum(zf, axis=0, keepdims=True))
            sqs.append(jnp.sum(zf * zf, axis=0, keepdims=True))
    if not finale:
        so_ref[...] = jnp.concatenate(sums + sqs, axis=0).reshape(so_ref.shape)


def _conv_layer(x, wlist, groups, couts, bt, bn_args=None, finale_bias=None):
    b, h, w, cin = x.shape
    n = b // bt
    r = b * h * w
    rt = bt * h * w
    G = len(groups)
    in_specs = [pl.BlockSpec((bt, h, w, cin), lambda i: (i, 0, 0, 0))]
    args = [x]
    bn = bn_args is not None
    if bn:
        stats_p, g_p, be_p, count_p, pairs_p = bn_args
        in_specs += [
            pl.BlockSpec(stats_p.shape, lambda i: (0, 0, 0)),
            pl.BlockSpec(g_p.shape, lambda i: (0, 0)),
            pl.BlockSpec(be_p.shape, lambda i: (0, 0)),
        ]
        args += [stats_p, g_p, be_p]
    else:
        count_p, pairs_p = 0.0, False
    finale = finale_bias is not None
    if finale:
        in_specs.append(pl.BlockSpec((1, 128), lambda i: (0, 0)))
        args.append(finale_bias)
    for wg in wlist:
        in_specs.append(pl.BlockSpec(wg.shape, lambda i: (0, 0, 0)))
        args.append(wg)
    out_shapes, out_specs = [], []
    for co in couts:
        out_shapes.append(jax.ShapeDtypeStruct(
            (r, co), jnp.float32 if finale else jnp.bfloat16))
        out_specs.append(pl.BlockSpec((rt, co), lambda i: (i, 0)))
    if not finale:
        L = couts[0]
        out_shapes.append(jax.ShapeDtypeStruct((n, 2 * G, L), jnp.float32))
        out_specs.append(pl.BlockSpec((1, 2 * G, L), lambda i: (i, 0, 0)))
    fn = functools.partial(_conv_kernel, bt, h, w, cin, groups, bn,
                           count_p, pairs_p, finale)
    return pl.pallas_call(
        fn,
        out_shape=tuple(out_shapes),
        grid=(n,),
        in_specs=in_specs,
        out_specs=tuple(out_specs),
        compiler_params=_CP,
    )(*args)


def _interleave(planes, b, h, w, c):
    c00, c01, c10, c11 = [p.reshape(b, h, w, c) for p in planes]
    even = jnp.stack([c00, c01], axis=3)
    odd = jnp.stack([c10, c11], axis=3)
    return jnp.stack([even, odd], axis=2).reshape(b, 2 * h, 2 * w, c)


def kernel(x, fc_w, fc_b,
           w1_0, w1_1, w1_2, w1_3, g1, beta1,
           w2_0, w2_1, w2_2, w2_3, g2, beta2,
           w3_0, w3_1, w3_2, w3_3, g3, beta3,
           w4_0, w4_1, w4_2, w4_3, b4):
    B = x.shape[0]
    h0 = jnp.dot(x, fc_w) + fc_b
    h0 = h0.reshape(B, 512, 8, 8).transpose(0, 2, 3, 1).astype(jnp.bfloat16)

    groups4 = tuple(_ptaps(a, b) for a, b in _PARITIES)

    w1 = [w.reshape(len(_ptaps(a, b)), 512, 256)
          for w, (a, b) in zip((w1_0, w1_1, w1_2, w1_3), _PARITIES)]
    *p1, stats1 = _conv_layer(h0, w1, groups4, (256,) * 4, bt=8)
    x1 = _interleave(p1, B, 8, 8, 256)

    w2 = [w.reshape(len(_ptaps(a, b)), 256, 128)
          for w, (a, b) in zip((w2_0, w2_1, w2_2, w2_3), _PARITIES)]
    *p2, stats2 = _conv_layer(
        x1, w2, groups4, (128,) * 4, bt=8,
        bn_args=(stats1, g1.reshape(1, -1), beta1.reshape(1, -1), 16384.0, False))
    x2 = _interleave(p2, B, 16, 16, 128)

    blocks3 = [_tap_blocks(w, a, b, 128)
               for w, (a, b) in zip((w3_0, w3_1, w3_2, w3_3), _PARITIES)]
    taps_a = _ALL_TAPS
    taps_b = tuple(t for t in _ALL_TAPS if t != (0, 0))
    wa = _group_weight([blocks3[0], blocks3[3]], taps_a, 128, 64)
    wb = _group_weight([blocks3[1], blocks3[2]], taps_b, 128, 64)
    za, zb, stats3 = _conv_layer(
        x2, [wa, wb], (taps_a, taps_b), (128, 128), bt=8,
        bn_args=(stats2, g2.reshape(1, -1), beta2.reshape(1, -1), 65536.0, False))
    p3 = [za[:, :64], zb[:, :64], zb[:, 64:], za[:, 64:]]
    x3 = _interleave(p3, B, 32, 32, 64)

    w4c = _w4_combined((w4_0, w4_1, w4_2, w4_3))
    bias4 = jnp.zeros((128,), jnp.float32).at[:12].set(
        jnp.repeat(b4[:3], 4)).reshape(1, 128)
    (z4,) = _conv_layer(
        x3, [w4c], (_ALL_TAPS,), (12,), bt=2,
        bn_args=(stats3, g3.reshape(1, -1), beta3.reshape(1, -1), 262144.0, True),
        finale_bias=bias4)
    out = (z4.reshape(B, 64, 64, 3, 2, 2)
           .transpose(0, 3, 1, 4, 2, 5)
           .reshape(B, 3, 128, 128))
    return out


# speedup vs baseline: 17.0066x; 1.7536x over previous
import functools

import jax
import jax.numpy as jnp
from jax.experimental import pallas as pl
from jax.experimental.pallas import tpu as pltpu

EPS = 1e-5
_PARITIES = ((0, 0), (0, 1), (1, 0), (1, 1))
_ALL_TAPS = tuple((dy, dx) for dy in (0, 1, 2) for dx in (0, 1, 2))

_CP = pltpu.CompilerParams(
    dimension_semantics=("parallel",),
    vmem_limit_bytes=48 * 1024 * 1024,
)


def _offs(p):
    return (0, 1, 2) if p == 0 else (1, 2)


def _ptaps(a, b):
    return tuple((dy, dx) for dy in _offs(a) for dx in _offs(b))


def _tap_blocks(w, a, b, cin):
    taps = _ptaps(a, b)
    wr = w.reshape(len(taps), cin, w.shape[-1])
    return {t: wr[i] for i, t in enumerate(taps)}


def _group_weight(blocks_list, taps, cin, cout_each):
    dt = next(iter(blocks_list[0].values())).dtype
    mats = []
    for t in taps:
        cols = [blocks.get(t, jnp.zeros((cin, cout_each), dt))
                for blocks in blocks_list]
        mats.append(jnp.concatenate(cols, axis=1) if len(cols) > 1 else cols[0])
    return jnp.stack(mats, axis=0)


def _w4_combined(w4s):
    blocks = [_tap_blocks(w4s[j], a, b, 64) for j, (a, b) in enumerate(_PARITIES)]
    mats = []
    for t in _ALL_TAPS:
        m = jnp.zeros((64, 128), jnp.bfloat16)
        for j in range(4):
            if t in blocks[j]:
                m = m.at[:, j:12:4].set(blocks[j][t][:, :3])
        mats.append(m)
    return jnp.stack(mats, axis=0)


def _conv_kernel(bt, h, w, cin, groups, bn, count_prev, pairs_prev, finale,
                 pair_split, *refs):
    G = len(groups)
    k = 0
    x_ref = refs[k]; k += 1
    if bn:
        stats_ref, g_ref, be_ref = refs[k], refs[k + 1], refs[k + 2]; k += 3
    if finale:
        bias_ref = refs[k]; k += 1
    w_refs = refs[k:k + G]; k += G
    n_out = G if finale else 1
    o_refs = refs[k:k + n_out]; k += n_out
    if not finale:
        so_ref = refs[k]; k += 1

    x = x_ref[...]
    if bn:
        tot = jnp.sum(stats_ref[...], axis=0)
        gp = tot.shape[0] // 2
        ssum = jnp.sum(tot[:gp], axis=0, keepdims=True)
        ssq = jnp.sum(tot[gp:], axis=0, keepdims=True)
        if pairs_prev:
            c = ssum.shape[1] // 2
            ssum = ssum[:, :c] + ssum[:, c:]
            ssq = ssq[:, :c] + ssq[:, c:]
        mean = ssum / count_prev
        var = jnp.maximum(ssq / count_prev - mean * mean, 0.0)
        scale = g_ref[...] * jax.lax.rsqrt(var + EPS)
        shift = be_ref[...] - mean * scale
        xf = (x.astype(jnp.float32) * scale.reshape(1, 1, 1, -1)
              + shift.reshape(1, 1, 1, -1))
        x = jnp.maximum(xf, 0.0).astype(jnp.bfloat16)

    zrow = jnp.zeros((bt, 1, w, cin), jnp.bfloat16)
    yp = jnp.concatenate([zrow, x, zrow], axis=1)
    zcol = jnp.zeros((bt, h + 2, 1, cin), jnp.bfloat16)
    yp = jnp.concatenate([zcol, yp, zcol], axis=2)

    r = bt * h * w
    sums, sqs, zcs = [], [], []
    for gi, taps in enumerate(groups):
        z = None
        for t, (dy, dx) in enumerate(taps):
            sl = yp[:, dy:dy + h, dx:dx + w, :].reshape(r, cin)
            zz = jnp.dot(sl, w_refs[gi][t], preferred_element_type=jnp.float32)
            z = zz if z is None else z + zz
        if finale:
            z = jnp.tanh(z + bias_ref[...])
            o_refs[gi][...] = z[:, :12]
        else:
            zc = z.astype(jnp.bfloat16)
            zcs.append(zc)
            zf = zc.astype(jnp.float32)
            sums.append(jnp.sum(zf, axis=0, keepdims=True))
            sqs.append(jnp.sum(zf * zf, axis=0, keepdims=True))
    if not finale:
        if pair_split:
            co = zcs[0].shape[1] // 2
            pv = (zcs[0][:, :co], zcs[1][:, :co],
                  zcs[1][:, co:], zcs[0][:, co:])
        else:
            co = zcs[0].shape[1]
            pv = zcs
        c00, c01, c10, c11 = [p.reshape(bt, h, w, co) for p in pv]
        even = jnp.stack([c00, c01], axis=3)
        odd = jnp.stack([c10, c11], axis=3)
        il = jnp.stack([even, odd], axis=2)
        o_refs[0][...] = il.reshape(bt, 2 * h, 2 * w, co)
        so_ref[...] = jnp.concatenate(sums + sqs, axis=0).reshape(so_ref.shape)


def _conv_layer(x, wlist, groups, couts, bt, bn_args=None, finale_bias=None,
                pair_split=False):
    b, h, w, cin = x.shape
    n = b // bt
    r = b * h * w
    rt = bt * h * w
    G = len(groups)
    in_specs = [pl.BlockSpec((bt, h, w, cin), lambda i: (i, 0, 0, 0))]
    args = [x]
    bn = bn_args is not None
    if bn:
        stats_p, g_p, be_p, count_p, pairs_p = bn_args
        in_specs += [
            pl.BlockSpec(stats_p.shape, lambda i: (0, 0, 0)),
            pl.BlockSpec(g_p.shape, lambda i: (0, 0)),
            pl.BlockSpec(be_p.shape, lambda i: (0, 0)),
        ]
        args += [stats_p, g_p, be_p]
    else:
        count_p, pairs_p = 0.0, False
    finale = finale_bias is not None
    if finale:
        in_specs.append(pl.BlockSpec((1, 128), lambda i: (0, 0)))
        args.append(finale_bias)
    for wg in wlist:
        in_specs.append(pl.BlockSpec(wg.shape, lambda i: (0, 0, 0)))
        args.append(wg)
    out_shapes, out_specs = [], []
    if finale:
        for co in couts:
            out_shapes.append(jax.ShapeDtypeStruct((r, co), jnp.float32))
            out_specs.append(pl.BlockSpec((rt, co), lambda i: (i, 0)))
    else:
        co = couts[0] // 2 if pair_split else couts[0]
        out_shapes.append(jax.ShapeDtypeStruct(
            (b, 2 * h, 2 * w, co), jnp.bfloat16))
        out_specs.append(pl.BlockSpec((bt, 2 * h, 2 * w, co),
                                      lambda i: (i, 0, 0, 0)))
        L = couts[0]
        out_shapes.append(jax.ShapeDtypeStruct((n, 2 * G, L), jnp.float32))
        out_specs.append(pl.BlockSpec((1, 2 * G, L), lambda i: (i, 0, 0)))
    fn = functools.partial(_conv_kernel, bt, h, w, cin, groups, bn,
                           count_p, pairs_p, finale, pair_split)
    return pl.pallas_call(
        fn,
        out_shape=tuple(out_shapes),
        grid=(n,),
        in_specs=in_specs,
        out_specs=tuple(out_specs),
        compiler_params=_CP,
    )(*args)


def kernel(x, fc_w, fc_b,
           w1_0, w1_1, w1_2, w1_3, g1, beta1,
           w2_0, w2_1, w2_2, w2_3, g2, beta2,
           w3_0, w3_1, w3_2, w3_3, g3, beta3,
           w4_0, w4_1, w4_2, w4_3, b4):
    B = x.shape[0]
    h0 = jnp.dot(x, fc_w) + fc_b
    h0 = h0.reshape(B, 512, 8, 8).transpose(0, 2, 3, 1).astype(jnp.bfloat16)

    groups4 = tuple(_ptaps(a, b) for a, b in _PARITIES)

    w1 = [w.reshape(len(_ptaps(a, b)), 512, 256)
          for w, (a, b) in zip((w1_0, w1_1, w1_2, w1_3), _PARITIES)]
    x1, stats1 = _conv_layer(h0, w1, groups4, (256,) * 4, bt=8)

    w2 = [w.reshape(len(_ptaps(a, b)), 256, 128)
          for w, (a, b) in zip((w2_0, w2_1, w2_2, w2_3), _PARITIES)]
    x2, stats2 = _conv_layer(
        x1, w2, groups4, (128,) * 4, bt=8,
        bn_args=(stats1, g1.reshape(1, -1), beta1.reshape(1, -1), 16384.0, False))

    blocks3 = [_tap_blocks(w, a, b, 128)
               for w, (a, b) in zip((w3_0, w3_1, w3_2, w3_3), _PARITIES)]
    taps_a = _ALL_TAPS
    taps_b = tuple(t for t in _ALL_TAPS if t != (0, 0))
    wa = _group_weight([blocks3[0], blocks3[3]], taps_a, 128, 64)
    wb = _group_weight([blocks3[1], blocks3[2]], taps_b, 128, 64)
    x3, stats3 = _conv_layer(
        x2, [wa, wb], (taps_a, taps_b), (128, 128), bt=8,
        bn_args=(stats2, g2.reshape(1, -1), beta2.reshape(1, -1), 65536.0, False),
        pair_split=True)

    w4c = _w4_combined((w4_0, w4_1, w4_2, w4_3))
    bias4 = jnp.zeros((128,), jnp.float32).at[:12].set(
        jnp.repeat(b4[:3], 4)).reshape(1, 128)
    (z4,) = _conv_layer(
        x3, [w4c], (_ALL_TAPS,), (12,), bt=2,
        bn_args=(stats3, g3.reshape(1, -1), beta3.reshape(1, -1), 262144.0, True),
        finale_bias=bias4)
    out = (z4.reshape(B, 64, 64, 3, 2, 2)
           .transpose(0, 3, 1, 4, 2, 5)
           .reshape(B, 3, 128, 128))
    return out
